# Initial kernel scaffold; baseline (speedup 1.0000x reference)
import jax
import jax.numpy as jnp
from jax.experimental import pallas as pl


def kernel(p, features, target):
    raise NotImplementedError("write your pallas kernel here")


# speedup vs baseline: 13.5166x; 13.5166x over previous
import jax
import jax.numpy as jnp
from jax.experimental import pallas as pl
from jax.experimental.pallas import tpu as pltpu

N = 10000
D = 64
NSAMPLE = 36
NUM_CLASSES = 13
TEMPERATURE = 0.1
WEIGHT = 1.0
EPS = 1e-7

Q = 80
GRID = N // Q
K = NSAMPLE - 1
NP = 10112
_INF = 3.0e38
_BIG = 1.0e38


def _block_kernel(p_blk, pT_all, f_blk, fT_all, t_col, t_row,
                  lsum_ref, nsel_ref, work_ref):
    i = pl.program_id(0)

    pb = p_blk[...]
    pa = pT_all[...]
    sq_b = jnp.sum(pb * pb, axis=1, keepdims=True)
    sq_a = jnp.sum(pa * pa, axis=0, keepdims=True)
    dot_p = jnp.dot(pb, pa, preferred_element_type=jnp.float32)
    d2 = sq_b + sq_a - 2.0 * dot_p
    work_ref[...] = d2

    colf = jax.lax.broadcasted_iota(jnp.int32, (Q, NP), 1).astype(jnp.float32)

    m1 = jnp.min(d2, axis=1, keepdims=True)
    c0 = jnp.min(jnp.where(d2 <= m1, colf, _BIG), axis=1, keepdims=True)

    def body(_, m_prev):
        w = work_ref[...]
        m = jnp.min(w, axis=1, keepdims=True)
        work_ref[...] = jnp.where(w <= m, _INF, w)
        return m

    jax.lax.fori_loop(0, NSAMPLE, body, jnp.zeros((Q, 1), jnp.float32))
    sel = jnp.logical_and(work_ref[...] >= _INF, colf != c0)

    fb = f_blk[...]
    fa = fT_all[...]
    fn_b = jnp.sum(fb * fb, axis=1, keepdims=True)
    fn_a = jnp.sum(fa * fa, axis=0, keepdims=True)
    dot_f = jnp.dot(fb, fa, preferred_element_type=jnp.float32,
                    precision=jax.lax.Precision.HIGHEST)
    fd2 = jnp.maximum(fn_b + fn_a - 2.0 * dot_f, 0.0)
    fdist = jnp.sqrt(fd2 + EPS)

    posmask = (t_col[...] == t_row[...])
    cnt = jnp.sum(jnp.where(sel & posmask, 1.0, 0.0), axis=1,
                  keepdims=True)

    mdist = jnp.min(jnp.where(sel, fdist, _INF), axis=1, keepdims=True)
    ex = jnp.exp(jnp.where(sel, (mdist - fdist) / TEMPERATURE, -1e9))
    pos = jnp.sum(jnp.where(posmask, ex, 0.0), axis=1, keepdims=True)
    neg = jnp.sum(ex, axis=1, keepdims=True)

    loss_i = -jnp.log(pos / neg + EPS)
    pm = jnp.logical_and(cnt > 0.0, cnt < float(K))
    part_sum = jnp.sum(jnp.where(pm, loss_i, 0.0)).reshape(1, 1)
    part_cnt = jnp.sum(jnp.where(pm, 1.0, 0.0)).reshape(1, 1)

    @pl.when(i == 0)
    def _init():
        lsum_ref[...] = jnp.zeros((1, 1), jnp.float32)
        nsel_ref[...] = jnp.zeros((1, 1), jnp.float32)

    lsum_ref[...] += part_sum
    nsel_ref[...] += part_cnt


@jax.jit
def kernel(p, features, target):
    pad = NP - N
    pT = jnp.pad(p.T, ((0, 0), (0, pad)), constant_values=1.0e17)
    fT = jnp.pad(features.T, ((0, 0), (0, pad)))
    t_col = target.reshape(N, 1)
    t_row = jnp.pad(target.reshape(1, N), ((0, 0), (0, pad)),
                    constant_values=-1)

    lsum, nsel = pl.pallas_call(
        _block_kernel,
        grid=(GRID,),
        in_specs=[
            pl.BlockSpec((Q, 3), lambda i: (i, 0)),
            pl.BlockSpec((3, NP), lambda i: (0, 0)),
            pl.BlockSpec((Q, D), lambda i: (i, 0)),
            pl.BlockSpec((D, NP), lambda i: (0, 0)),
            pl.BlockSpec((Q, 1), lambda i: (i, 0)),
            pl.BlockSpec((1, NP), lambda i: (0, 0)),
        ],
        out_specs=[
            pl.BlockSpec((1, 1), lambda i: (0, 0)),
            pl.BlockSpec((1, 1), lambda i: (0, 0)),
        ],
        out_shape=[
            jax.ShapeDtypeStruct((1, 1), jnp.float32),
            jax.ShapeDtypeStruct((1, 1), jnp.float32),
        ],
        scratch_shapes=[
            pltpu.VMEM((Q, NP), jnp.float32),
        ],
    )(p, pT, features, fT, t_col, t_row)

    loss = lsum[0, 0] / nsel[0, 0]
    loss = jnp.where(jnp.isfinite(loss), loss, jnp.zeros_like(loss))
    return loss * WEIGHT


# speedup vs baseline: 20.6897x; 1.5307x over previous
import jax
import jax.numpy as jnp
from jax.experimental import pallas as pl
from jax.experimental.pallas import tpu as pltpu

N = 10000
D = 64
NSAMPLE = 36
NUM_CLASSES = 13
TEMPERATURE = 0.1
WEIGHT = 1.0
EPS = 1e-7

Q = 80
GRID = N // Q
K = NSAMPLE - 1
NP = 10240
S = 8
G = NP // S
_INF = 3.0e38
_BIG = 1.0e38
_NEG = -3.0e38


def _block_kernel(p_blk, pT_all, f_blk, fT_all, t_col, t_row,
                  lsum_ref, nsel_ref, d2_ref, gm_ref):
    i = pl.program_id(0)

    pb = p_blk[...]
    pa = pT_all[...]
    sq_b = jnp.sum(pb * pb, axis=1, keepdims=True)
    sq_a = jnp.sum(pa * pa, axis=0, keepdims=True)
    dot_p = jnp.dot(pb, pa, preferred_element_type=jnp.float32)
    d2 = sq_b + sq_a - 2.0 * dot_p
    d2_ref[...] = d2

    colf = jax.lax.broadcasted_iota(jnp.int32, (Q, NP), 1).astype(jnp.float32)

    m1 = jnp.min(d2, axis=1, keepdims=True)
    c0 = jnp.min(jnp.where(d2 <= m1, colf, _BIG), axis=1, keepdims=True)

    gm = d2[:, 0:G]
    for s in range(1, S):
        gm = jnp.minimum(gm, d2[:, s * G:(s + 1) * G])
    gm_ref[...] = gm

    def body(_, m_prev):
        w = gm_ref[...]
        m = jnp.min(w, axis=1, keepdims=True)
        gm_ref[...] = jnp.where(w <= m, _INF, w)
        return m

    t_ub = jax.lax.fori_loop(0, NSAMPLE, body,
                             jnp.zeros((Q, 1), jnp.float32))

    seld2 = jnp.where(d2_ref[...] <= t_ub, d2_ref[...], _NEG)
    d2_ref[...] = seld2
    s_cnt = jnp.sum(jnp.where(seld2 > _NEG, 1.0, 0.0), axis=1,
                    keepdims=True)

    def trim_cond(s_c):
        return jnp.any(s_c > float(NSAMPLE))

    def trim_body(s_c):
        sd = d2_ref[...]
        need = s_c > float(NSAMPLE)
        mx = jnp.max(sd, axis=1, keepdims=True)
        remove = jnp.logical_and(need, sd >= mx)
        d2_ref[...] = jnp.where(remove, _NEG, sd)
        return s_c - jnp.sum(jnp.where(remove, 1.0, 0.0), axis=1,
                             keepdims=True)

    jax.lax.while_loop(trim_cond, trim_body, s_cnt)
    sel = jnp.logical_and(d2_ref[...] > _NEG, colf != c0)

    fb = f_blk[...]
    fa = fT_all[...]
    fn_b = jnp.sum(fb * fb, axis=1, keepdims=True)
    fn_a = jnp.sum(fa * fa, axis=0, keepdims=True)
    dot_f = jnp.dot(fb, fa, preferred_element_type=jnp.float32,
                    precision=jax.lax.Precision.HIGHEST)
    fd2 = jnp.maximum(fn_b + fn_a - 2.0 * dot_f, 0.0)
    fdist = jnp.sqrt(fd2 + EPS)

    posmask = (t_col[...] == t_row[...])
    cnt = jnp.sum(jnp.where(sel & posmask, 1.0, 0.0), axis=1,
                  keepdims=True)

    mdist = jnp.min(jnp.where(sel, fdist, _INF), axis=1, keepdims=True)
    ex = jnp.exp(jnp.where(sel, (mdist - fdist) / TEMPERATURE, -1e9))
    pos = jnp.sum(jnp.where(posmask, ex, 0.0), axis=1, keepdims=True)
    neg = jnp.sum(ex, axis=1, keepdims=True)

    loss_i = -jnp.log(pos / neg + EPS)
    pm = jnp.logical_and(cnt > 0.0, cnt < float(K))
    part_sum = jnp.sum(jnp.where(pm, loss_i, 0.0)).reshape(1, 1)
    part_cnt = jnp.sum(jnp.where(pm, 1.0, 0.0)).reshape(1, 1)

    @pl.when(i == 0)
    def _init():
        lsum_ref[...] = jnp.zeros((1, 1), jnp.float32)
        nsel_ref[...] = jnp.zeros((1, 1), jnp.float32)

    lsum_ref[...] += part_sum
    nsel_ref[...] += part_cnt


@jax.jit
def kernel(p, features, target):
    pad = NP - N
    pT = jnp.pad(p.T, ((0, 0), (0, pad)), constant_values=1.0e17)
    fT = jnp.pad(features.T, ((0, 0), (0, pad)))
    t_col = target.reshape(N, 1)
    t_row = jnp.pad(target.reshape(1, N), ((0, 0), (0, pad)),
                    constant_values=-1)

    lsum, nsel = pl.pallas_call(
        _block_kernel,
        grid=(GRID,),
        in_specs=[
            pl.BlockSpec((Q, 3), lambda i: (i, 0)),
            pl.BlockSpec((3, NP), lambda i: (0, 0)),
            pl.BlockSpec((Q, D), lambda i: (i, 0)),
            pl.BlockSpec((D, NP), lambda i: (0, 0)),
            pl.BlockSpec((Q, 1), lambda i: (i, 0)),
            pl.BlockSpec((1, NP), lambda i: (0, 0)),
        ],
        out_specs=[
            pl.BlockSpec((1, 1), lambda i: (0, 0)),
            pl.BlockSpec((1, 1), lambda i: (0, 0)),
        ],
        out_shape=[
            jax.ShapeDtypeStruct((1, 1), jnp.float32),
            jax.ShapeDtypeStruct((1, 1), jnp.float32),
        ],
        scratch_shapes=[
            pltpu.VMEM((Q, NP), jnp.float32),
            pltpu.VMEM((Q, G), jnp.float32),
        ],
    )(p, pT, features, fT, t_col, t_row)

    loss = lsum[0, 0] / nsel[0, 0]
    loss = jnp.where(jnp.isfinite(loss), loss, jnp.zeros_like(loss))
    return loss * WEIGHT


# speedup vs baseline: 27.1122x; 1.3104x over previous
import jax
import jax.numpy as jnp
from jax.experimental import pallas as pl
from jax.experimental.pallas import tpu as pltpu

N = 10000
D = 64
NSAMPLE = 36
NUM_CLASSES = 13
TEMPERATURE = 0.1
WEIGHT = 1.0
EPS = 1e-7

Q = 200
GRID = N // Q
K = NSAMPLE - 1
NP = 10240
S = 8
G = NP // S
_INF = 3.0e38
_BIG = 1.0e38
_NEG = -3.0e38


def _block_kernel(p_blk, pT_all, f_blk, fT_all, t_col, t_row,
                  lsum_ref, nsel_ref, d2_ref, gm_ref):
    i = pl.program_id(0)

    pb = p_blk[...]
    pa = pT_all[...]
    sq_b = jnp.sum(pb * pb, axis=1, keepdims=True)
    sq_a = jnp.sum(pa * pa, axis=0, keepdims=True)
    dot_p = jnp.dot(pb, pa, preferred_element_type=jnp.float32)
    d2 = sq_b + sq_a - 2.0 * dot_p
    d2_ref[...] = d2

    colf = jax.lax.broadcasted_iota(jnp.int32, (Q, NP), 1).astype(jnp.float32)

    m1 = jnp.min(d2, axis=1, keepdims=True)
    c0 = jnp.min(jnp.where(d2 <= m1, colf, _BIG), axis=1, keepdims=True)

    gm = d2[:, 0:G]
    for s in range(1, S):
        gm = jnp.minimum(gm, d2[:, s * G:(s + 1) * G])
    gm_ref[...] = gm

    def body(_, m_prev):
        w = gm_ref[...]
        m = jnp.min(w, axis=1, keepdims=True)
        gm_ref[...] = jnp.where(w <= m, _INF, w)
        return m

    t_ub = jax.lax.fori_loop(0, NSAMPLE, body,
                             jnp.zeros((Q, 1), jnp.float32))

    seld2 = jnp.where(d2_ref[...] <= t_ub, d2_ref[...], _NEG)
    d2_ref[...] = seld2
    s_cnt = jnp.sum(jnp.where(seld2 > _NEG, 1.0, 0.0), axis=1,
                    keepdims=True)

    def trim_cond(s_c):
        return jnp.any(s_c > float(NSAMPLE))

    def trim_body(s_c):
        sd = d2_ref[...]
        need = s_c > float(NSAMPLE)
        mx = jnp.max(sd, axis=1, keepdims=True)
        remove = jnp.logical_and(need, sd >= mx)
        d2_ref[...] = jnp.where(remove, _NEG, sd)
        return s_c - jnp.sum(jnp.where(remove, 1.0, 0.0), axis=1,
                             keepdims=True)

    jax.lax.while_loop(trim_cond, trim_body, s_cnt)
    sel = jnp.logical_and(d2_ref[...] > _NEG, colf != c0)

    fb = f_blk[...]
    fa = fT_all[...]
    fn_b = jnp.sum(fb * fb, axis=1, keepdims=True)
    fn_a = jnp.sum(fa * fa, axis=0, keepdims=True)
    dot_f = jnp.dot(fb, fa, preferred_element_type=jnp.float32)
    fd2 = jnp.maximum(fn_b + fn_a - 2.0 * dot_f, 0.0)
    fdist = jnp.sqrt(fd2 + EPS)

    posmask = (t_col[...] == t_row[...])
    cnt = jnp.sum(jnp.where(sel & posmask, 1.0, 0.0), axis=1,
                  keepdims=True)

    mdist = jnp.min(jnp.where(sel, fdist, _INF), axis=1, keepdims=True)
    ex = jnp.exp(jnp.where(sel, (mdist - fdist) / TEMPERATURE, -1e9))
    pos = jnp.sum(jnp.where(posmask, ex, 0.0), axis=1, keepdims=True)
    neg = jnp.sum(ex, axis=1, keepdims=True)

    loss_i = -jnp.log(pos / neg + EPS)
    pm = jnp.logical_and(cnt > 0.0, cnt < float(K))
    part_sum = jnp.sum(jnp.where(pm, loss_i, 0.0)).reshape(1, 1)
    part_cnt = jnp.sum(jnp.where(pm, 1.0, 0.0)).reshape(1, 1)

    @pl.when(i == 0)
    def _init():
        lsum_ref[...] = jnp.zeros((1, 1), jnp.float32)
        nsel_ref[...] = jnp.zeros((1, 1), jnp.float32)

    lsum_ref[...] += part_sum
    nsel_ref[...] += part_cnt


@jax.jit
def kernel(p, features, target):
    pad = NP - N
    pT = jnp.pad(p.T, ((0, 0), (0, pad)), constant_values=1.0e17)
    fT = jnp.pad(features.T, ((0, 0), (0, pad)))
    t_col = target.reshape(N, 1)
    t_row = jnp.pad(target.reshape(1, N), ((0, 0), (0, pad)),
                    constant_values=-1)

    lsum, nsel = pl.pallas_call(
        _block_kernel,
        grid=(GRID,),
        in_specs=[
            pl.BlockSpec((Q, 3), lambda i: (i, 0)),
            pl.BlockSpec((3, NP), lambda i: (0, 0)),
            pl.BlockSpec((Q, D), lambda i: (i, 0)),
            pl.BlockSpec((D, NP), lambda i: (0, 0)),
            pl.BlockSpec((Q, 1), lambda i: (i, 0)),
            pl.BlockSpec((1, NP), lambda i: (0, 0)),
        ],
        out_specs=[
            pl.BlockSpec((1, 1), lambda i: (0, 0)),
            pl.BlockSpec((1, 1), lambda i: (0, 0)),
        ],
        out_shape=[
            jax.ShapeDtypeStruct((1, 1), jnp.float32),
            jax.ShapeDtypeStruct((1, 1), jnp.float32),
        ],
        scratch_shapes=[
            pltpu.VMEM((Q, NP), jnp.float32),
            pltpu.VMEM((Q, G), jnp.float32),
        ],
    )(p, pT, features, fT, t_col, t_row)

    loss = lsum[0, 0] / nsel[0, 0]
    loss = jnp.where(jnp.isfinite(loss), loss, jnp.zeros_like(loss))
    return loss * WEIGHT


# speedup vs baseline: 30.2779x; 1.1168x over previous
import jax
import jax.numpy as jnp
from jax.experimental import pallas as pl
from jax.experimental.pallas import tpu as pltpu

N = 10000
D = 64
NSAMPLE = 36
NUM_CLASSES = 13
TEMPERATURE = 0.1
WEIGHT = 1.0
EPS = 1e-7

Q = 200
GRID = N // Q
K = NSAMPLE - 1
NP = 10240
S = 8
G = NP // S
_INF = 3.0e38
_BIG = 1.0e38
_NEG = -3.0e38


def _block_kernel(p_blk, pT_all, f_blk, fT_all, t_col, t_row,
                  lsum_ref, nsel_ref, d2_ref, gm_ref):
    i = pl.program_id(0)

    pb = p_blk[...]
    pa = pT_all[...]
    sq_b = jnp.sum(pb * pb, axis=1, keepdims=True)
    sq_a = jnp.sum(pa * pa, axis=0, keepdims=True)
    dot_p = jnp.dot(pb, pa, preferred_element_type=jnp.float32)
    d2 = sq_b + sq_a - 2.0 * dot_p
    d2_ref[...] = d2

    colf = jax.lax.broadcasted_iota(jnp.int32, (Q, NP), 1).astype(jnp.float32)

    m1 = jnp.min(d2, axis=1, keepdims=True)
    c0 = jnp.min(jnp.where(d2 <= m1, colf, _BIG), axis=1, keepdims=True)

    mn = d2[:, 0:G]
    mn2 = jnp.full_like(mn, _INF)
    for s in range(1, S):
        x = d2[:, s * G:(s + 1) * G]
        mn2 = jnp.minimum(mn2, jnp.maximum(mn, x))
        mn = jnp.minimum(mn, x)
    gm_ref[:, 0:G] = mn
    gm_ref[:, G:2 * G] = mn2

    def body(_, m_prev):
        w = gm_ref[...]
        m = jnp.min(w, axis=1, keepdims=True)
        gm_ref[...] = jnp.where(w <= m, _INF, w)
        return m

    t_ub = jax.lax.fori_loop(0, NSAMPLE, body,
                             jnp.zeros((Q, 1), jnp.float32))

    seld2 = jnp.where(d2_ref[...] <= t_ub, d2_ref[...], _NEG)
    d2_ref[...] = seld2
    s_cnt = jnp.sum(jnp.where(seld2 > _NEG, 1.0, 0.0), axis=1,
                    keepdims=True)

    def trim_cond(s_c):
        return jnp.any(s_c > float(NSAMPLE))

    def trim_body(s_c):
        sd = d2_ref[...]
        need = s_c > float(NSAMPLE)
        mx = jnp.max(sd, axis=1, keepdims=True)
        remove = jnp.logical_and(need, sd >= mx)
        d2_ref[...] = jnp.where(remove, _NEG, sd)
        return s_c - jnp.sum(jnp.where(remove, 1.0, 0.0), axis=1,
                             keepdims=True)

    jax.lax.while_loop(trim_cond, trim_body, s_cnt)
    sel = jnp.logical_and(d2_ref[...] > _NEG, colf != c0)

    fb = f_blk[...]
    fa = fT_all[...]
    fn_b = jnp.sum(fb * fb, axis=1, keepdims=True)
    fn_a = jnp.sum(fa * fa, axis=0, keepdims=True)
    dot_f = jnp.dot(fb, fa, preferred_element_type=jnp.float32)
    fd2 = jnp.maximum(fn_b + fn_a - 2.0 * dot_f, 0.0)
    fdist = jnp.sqrt(fd2 + EPS)

    posmask = (t_col[...] == t_row[...])
    cnt = jnp.sum(jnp.where(sel & posmask, 1.0, 0.0), axis=1,
                  keepdims=True)

    mdist = jnp.min(jnp.where(sel, fdist, _INF), axis=1, keepdims=True)
    ex = jnp.exp(jnp.where(sel, (mdist - fdist) / TEMPERATURE, -1e9))
    pos = jnp.sum(jnp.where(posmask, ex, 0.0), axis=1, keepdims=True)
    neg = jnp.sum(ex, axis=1, keepdims=True)

    loss_i = -jnp.log(pos / neg + EPS)
    pm = jnp.logical_and(cnt > 0.0, cnt < float(K))
    part_sum = jnp.sum(jnp.where(pm, loss_i, 0.0)).reshape(1, 1)
    part_cnt = jnp.sum(jnp.where(pm, 1.0, 0.0)).reshape(1, 1)

    @pl.when(i == 0)
    def _init():
        lsum_ref[...] = jnp.zeros((1, 1), jnp.float32)
        nsel_ref[...] = jnp.zeros((1, 1), jnp.float32)

    lsum_ref[...] += part_sum
    nsel_ref[...] += part_cnt


@jax.jit
def kernel(p, features, target):
    pad = NP - N
    pT = jnp.pad(p.T, ((0, 0), (0, pad)), constant_values=1.0e17)
    fT = jnp.pad(features.T, ((0, 0), (0, pad)))
    t_col = target.reshape(N, 1)
    t_row = jnp.pad(target.reshape(1, N), ((0, 0), (0, pad)),
                    constant_values=-1)

    lsum, nsel = pl.pallas_call(
        _block_kernel,
        grid=(GRID,),
        in_specs=[
            pl.BlockSpec((Q, 3), lambda i: (i, 0)),
            pl.BlockSpec((3, NP), lambda i: (0, 0)),
            pl.BlockSpec((Q, D), lambda i: (i, 0)),
            pl.BlockSpec((D, NP), lambda i: (0, 0)),
            pl.BlockSpec((Q, 1), lambda i: (i, 0)),
            pl.BlockSpec((1, NP), lambda i: (0, 0)),
        ],
        out_specs=[
            pl.BlockSpec((1, 1), lambda i: (0, 0)),
            pl.BlockSpec((1, 1), lambda i: (0, 0)),
        ],
        out_shape=[
            jax.ShapeDtypeStruct((1, 1), jnp.float32),
            jax.ShapeDtypeStruct((1, 1), jnp.float32),
        ],
        scratch_shapes=[
            pltpu.VMEM((Q, NP), jnp.float32),
            pltpu.VMEM((Q, 2 * G), jnp.float32),
        ],
    )(p, pT, features, fT, t_col, t_row)

    loss = lsum[0, 0] / nsel[0, 0]
    loss = jnp.where(jnp.isfinite(loss), loss, jnp.zeros_like(loss))
    return loss * WEIGHT


# speedup vs baseline: 32.3596x; 1.0688x over previous
import jax
import jax.numpy as jnp
from jax.experimental import pallas as pl
from jax.experimental.pallas import tpu as pltpu

N = 10000
D = 64
NSAMPLE = 36
NUM_CLASSES = 13
TEMPERATURE = 0.1
WEIGHT = 1.0
EPS = 1e-7

Q = 200
GRID = N // Q
K = NSAMPLE - 1
NP = 10240
S = 8
G = NP // S
_INF = 3.0e38
_BIG = 1.0e38
_NEG = -3.0e38


def _block_kernel(p_blk, pT_all, f_blk, fT_all, t_col, t_row,
                  lsum_ref, nsel_ref, d2_ref, gm_ref):
    i = pl.program_id(0)

    pb = p_blk[...]
    pa = pT_all[...]
    sq_b = jnp.sum(pb * pb, axis=1, keepdims=True)
    sq_a = jnp.sum(pa * pa, axis=0, keepdims=True)
    dot_p = jnp.dot(pb, pa, preferred_element_type=jnp.float32)
    d2 = sq_b + sq_a - 2.0 * dot_p
    d2_ref[...] = d2

    colf = jax.lax.broadcasted_iota(jnp.int32, (Q, NP), 1).astype(jnp.float32)

    m1 = jnp.min(d2, axis=1, keepdims=True)
    c0 = jnp.min(jnp.where(d2 <= m1, colf, _BIG), axis=1, keepdims=True)

    mn = d2[:, 0:G]
    mn2 = jnp.full_like(mn, _INF)
    for s in range(1, S):
        x = d2[:, s * G:(s + 1) * G]
        mn2 = jnp.minimum(mn2, jnp.maximum(mn, x))
        mn = jnp.minimum(mn, x)
    gm_ref[:, 0:G] = mn
    gm_ref[:, G:2 * G] = mn2

    m_real = jnp.where(gm_ref[...] < 1e30, gm_ref[...], _NEG)
    hi0 = jnp.max(m_real, axis=1, keepdims=True)

    def bisect(_, carry):
        lo, hi = carry
        mid = 0.5 * (lo + hi)
        c = jnp.sum(jnp.where(gm_ref[...] <= mid, 1.0, 0.0), axis=1,
                    keepdims=True)
        pred = c >= float(NSAMPLE)
        return jnp.where(pred, lo, mid), jnp.where(pred, mid, hi)

    _, t_ub = jax.lax.fori_loop(0, 28, bisect, (m1, hi0))

    seld2 = jnp.where(d2_ref[...] <= t_ub, d2_ref[...], _NEG)
    d2_ref[...] = seld2
    s_cnt = jnp.sum(jnp.where(seld2 > _NEG, 1.0, 0.0), axis=1,
                    keepdims=True)

    def trim_cond(s_c):
        return jnp.any(s_c > float(NSAMPLE))

    def trim_body(s_c):
        sd = d2_ref[...]
        need = s_c > float(NSAMPLE)
        mx = jnp.max(sd, axis=1, keepdims=True)
        remove = jnp.logical_and(need, sd >= mx)
        d2_ref[...] = jnp.where(remove, _NEG, sd)
        return s_c - jnp.sum(jnp.where(remove, 1.0, 0.0), axis=1,
                             keepdims=True)

    jax.lax.while_loop(trim_cond, trim_body, s_cnt)
    sel = jnp.logical_and(d2_ref[...] > _NEG, colf != c0)

    fb = f_blk[...]
    fa = fT_all[...]
    fn_b = jnp.sum(fb * fb, axis=1, keepdims=True)
    fn_a = jnp.sum(fa * fa, axis=0, keepdims=True)
    dot_f = jnp.dot(fb, fa, preferred_element_type=jnp.float32)
    fd2 = jnp.maximum(fn_b + fn_a - 2.0 * dot_f, 0.0)
    fdist = jnp.sqrt(fd2 + EPS)

    posmask = (t_col[...] == t_row[...])
    cnt = jnp.sum(jnp.where(sel & posmask, 1.0, 0.0), axis=1,
                  keepdims=True)

    mdist = jnp.min(jnp.where(sel, fdist, _INF), axis=1, keepdims=True)
    ex = jnp.exp(jnp.where(sel, (mdist - fdist) / TEMPERATURE, -1e9))
    pos = jnp.sum(jnp.where(posmask, ex, 0.0), axis=1, keepdims=True)
    neg = jnp.sum(ex, axis=1, keepdims=True)

    loss_i = -jnp.log(pos / neg + EPS)
    pm = jnp.logical_and(cnt > 0.0, cnt < float(K))
    part_sum = jnp.sum(jnp.where(pm, loss_i, 0.0)).reshape(1, 1)
    part_cnt = jnp.sum(jnp.where(pm, 1.0, 0.0)).reshape(1, 1)

    @pl.when(i == 0)
    def _init():
        lsum_ref[...] = jnp.zeros((1, 1), jnp.float32)
        nsel_ref[...] = jnp.zeros((1, 1), jnp.float32)

    lsum_ref[...] += part_sum
    nsel_ref[...] += part_cnt


@jax.jit
def kernel(p, features, target):
    pad = NP - N
    pT = jnp.pad(p.T, ((0, 0), (0, pad)), constant_values=1.0e17)
    fT = jnp.pad(features.T, ((0, 0), (0, pad)))
    t_col = target.reshape(N, 1)
    t_row = jnp.pad(target.reshape(1, N), ((0, 0), (0, pad)),
                    constant_values=-1)

    lsum, nsel = pl.pallas_call(
        _block_kernel,
        grid=(GRID,),
        in_specs=[
            pl.BlockSpec((Q, 3), lambda i: (i, 0)),
            pl.BlockSpec((3, NP), lambda i: (0, 0)),
            pl.BlockSpec((Q, D), lambda i: (i, 0)),
            pl.BlockSpec((D, NP), lambda i: (0, 0)),
            pl.BlockSpec((Q, 1), lambda i: (i, 0)),
            pl.BlockSpec((1, NP), lambda i: (0, 0)),
        ],
        out_specs=[
            pl.BlockSpec((1, 1), lambda i: (0, 0)),
            pl.BlockSpec((1, 1), lambda i: (0, 0)),
        ],
        out_shape=[
            jax.ShapeDtypeStruct((1, 1), jnp.float32),
            jax.ShapeDtypeStruct((1, 1), jnp.float32),
        ],
        scratch_shapes=[
            pltpu.VMEM((Q, NP), jnp.float32),
            pltpu.VMEM((Q, 2 * G), jnp.float32),
        ],
    )(p, pT, features, fT, t_col, t_row)

    loss = lsum[0, 0] / nsel[0, 0]
    loss = jnp.where(jnp.isfinite(loss), loss, jnp.zeros_like(loss))
    return loss * WEIGHT


# speedup vs baseline: 33.9345x; 1.0487x over previous
import jax
import jax.numpy as jnp
from jax.experimental import pallas as pl
from jax.experimental.pallas import tpu as pltpu

N = 10000
D = 64
NSAMPLE = 36
NUM_CLASSES = 13
TEMPERATURE = 0.1
WEIGHT = 1.0
EPS = 1e-7

Q = 200
GRID = N // Q
K = NSAMPLE - 1
NP = 10240
S = 8
G = NP // S
_INF = 3.0e38
_BIG = 1.0e38
_NEG = -3.0e38


def _block_kernel(p_blk, pT_all, f_blk, fT_all, t_col, t_row,
                  lsum_ref, nsel_ref, d2_ref, gm_ref):
    i = pl.program_id(0)

    pb = p_blk[...]
    pa = pT_all[...]
    sq_b = jnp.sum(pb * pb, axis=1, keepdims=True)
    sq_a = jnp.sum(pa * pa, axis=0, keepdims=True)
    dot_p = jnp.dot(pb, pa, preferred_element_type=jnp.float32)
    d2 = sq_b + sq_a - 2.0 * dot_p
    d2_ref[...] = d2

    colf = jax.lax.broadcasted_iota(jnp.int32, (Q, NP), 1).astype(jnp.float32)

    mn = d2[:, 0:G]
    mn2 = jnp.full_like(mn, _INF)
    for s in range(1, S):
        x = d2[:, s * G:(s + 1) * G]
        mn2 = jnp.minimum(mn2, jnp.maximum(mn, x))
        mn = jnp.minimum(mn, x)
    gm_ref[:, 0:G] = mn
    gm_ref[:, G:2 * G] = mn2

    m1 = jnp.min(mn, axis=1, keepdims=True)
    c0 = jnp.min(jnp.where(d2 <= m1, colf, _BIG), axis=1, keepdims=True)

    m_real = jnp.where(gm_ref[...] < 1e30, gm_ref[...], _NEG)
    hi0 = jnp.max(m_real, axis=1, keepdims=True)

    def bisect(_, carry):
        lo, hi = carry
        mid = 0.5 * (lo + hi)
        c = jnp.sum(jnp.where(gm_ref[...] <= mid, 1.0, 0.0), axis=1,
                    keepdims=True)
        pred = c >= float(NSAMPLE)
        return jnp.where(pred, lo, mid), jnp.where(pred, mid, hi)

    _, t_ub = jax.lax.fori_loop(0, 28, bisect, (m1, hi0))

    s_cnt = jnp.sum(jnp.where(d2_ref[...] <= t_ub, 1.0, 0.0), axis=1,
                    keepdims=True)

    def trim_cond(carry):
        _, s_c = carry
        return jnp.any(s_c > float(NSAMPLE))

    def trim_body(carry):
        t, s_c = carry
        dd = d2_ref[...]
        need = s_c > float(NSAMPLE)
        cand = jnp.where(dd <= t, dd, _NEG)
        mx = jnp.max(cand, axis=1, keepdims=True)
        t2 = jnp.max(jnp.where(cand < mx, cand, _NEG), axis=1,
                     keepdims=True)
        t_new = jnp.where(need, t2, t)
        s_new = jnp.sum(jnp.where(dd <= t_new, 1.0, 0.0), axis=1,
                        keepdims=True)
        return t_new, s_new

    t_ub, _ = jax.lax.while_loop(trim_cond, trim_body, (t_ub, s_cnt))
    sel = jnp.logical_and(d2_ref[...] <= t_ub, colf != c0)

    fb = f_blk[...]
    fa = fT_all[...]
    fn_b = jnp.sum(fb * fb, axis=1, keepdims=True)
    fn_a = jnp.sum(fa * fa, axis=0, keepdims=True)
    dot_f = jnp.dot(fb, fa, preferred_element_type=jnp.float32)
    fd2 = jnp.maximum(fn_b + fn_a - 2.0 * dot_f, 0.0)
    fdist = jnp.sqrt(fd2 + EPS)

    posmask = (t_col[...] == t_row[...])
    cnt = jnp.sum(jnp.where(sel & posmask, 1.0, 0.0), axis=1,
                  keepdims=True)

    mdist = jnp.min(jnp.where(sel, fdist, _INF), axis=1, keepdims=True)
    ex = jnp.exp(jnp.where(sel, (mdist - fdist) / TEMPERATURE, -1e9))
    pos = jnp.sum(jnp.where(posmask, ex, 0.0), axis=1, keepdims=True)
    neg = jnp.sum(ex, axis=1, keepdims=True)

    loss_i = -jnp.log(pos / neg + EPS)
    pm = jnp.logical_and(cnt > 0.0, cnt < float(K))
    part_sum = jnp.sum(jnp.where(pm, loss_i, 0.0)).reshape(1, 1)
    part_cnt = jnp.sum(jnp.where(pm, 1.0, 0.0)).reshape(1, 1)

    @pl.when(i == 0)
    def _init():
        lsum_ref[...] = jnp.zeros((1, 1), jnp.float32)
        nsel_ref[...] = jnp.zeros((1, 1), jnp.float32)

    lsum_ref[...] += part_sum
    nsel_ref[...] += part_cnt


@jax.jit
def kernel(p, features, target):
    pad = NP - N
    pT = jnp.pad(p.T, ((0, 0), (0, pad)), constant_values=1.0e17)
    fT = jnp.pad(features.T, ((0, 0), (0, pad)))
    t_col = target.reshape(N, 1)
    t_row = jnp.pad(target.reshape(1, N), ((0, 0), (0, pad)),
                    constant_values=-1)

    lsum, nsel = pl.pallas_call(
        _block_kernel,
        grid=(GRID,),
        in_specs=[
            pl.BlockSpec((Q, 3), lambda i: (i, 0)),
            pl.BlockSpec((3, NP), lambda i: (0, 0)),
            pl.BlockSpec((Q, D), lambda i: (i, 0)),
            pl.BlockSpec((D, NP), lambda i: (0, 0)),
            pl.BlockSpec((Q, 1), lambda i: (i, 0)),
            pl.BlockSpec((1, NP), lambda i: (0, 0)),
        ],
        out_specs=[
            pl.BlockSpec((1, 1), lambda i: (0, 0)),
            pl.BlockSpec((1, 1), lambda i: (0, 0)),
        ],
        out_shape=[
            jax.ShapeDtypeStruct((1, 1), jnp.float32),
            jax.ShapeDtypeStruct((1, 1), jnp.float32),
        ],
        scratch_shapes=[
            pltpu.VMEM((Q, NP), jnp.float32),
            pltpu.VMEM((Q, 2 * G), jnp.float32),
        ],
    )(p, pT, features, fT, t_col, t_row)

    loss = lsum[0, 0] / nsel[0, 0]
    loss = jnp.where(jnp.isfinite(loss), loss, jnp.zeros_like(loss))
    return loss * WEIGHT
